# Initial kernel scaffold; baseline (speedup 1.0000x reference)
import jax
import jax.numpy as jnp
from jax.experimental import pallas as pl


def kernel(e, table):
    raise NotImplementedError("write your pallas kernel here")


# speedup vs baseline: 2.4899x; 2.4899x over previous
import functools

import jax
import jax.numpy as jnp
from jax import lax
from jax.experimental import pallas as pl
from jax.experimental.pallas import tpu as pltpu
from jax.experimental.pallas import tpu_sc as plsc

_NC = 2
_NS = 16
_NW = _NC * _NS

_CHUNK = 2048


def _emb_body(table_hbm, idx_hbm, out_hbm, idx_v, rows_v, sem):
    wid = lax.axis_index("s") * _NC + lax.axis_index("c")
    b_per_w = idx_hbm.shape[0] // _NW
    base = wid * b_per_w
    nchunks = b_per_w // _CHUNK

    def step(i, carry):
        off = base + i * _CHUNK
        pltpu.sync_copy(idx_hbm.at[pl.ds(off, _CHUNK)], idx_v)
        pltpu.async_copy(table_hbm.at[idx_v], rows_v, sem).wait()
        pltpu.sync_copy(rows_v, out_hbm.at[pl.ds(off, _CHUNK)])
        return carry

    lax.fori_loop(0, nchunks, step, 0)


def kernel(e, table):
    batch, hist = e.shape
    vocab, d = table.shape
    b_total = batch * hist
    assert b_total % (_NW * _CHUNK) == 0

    idx = e.reshape(b_total).astype(jnp.int32)

    mesh = plsc.VectorSubcoreMesh(core_axis_name="c", subcore_axis_name="s")
    run = pl.kernel(
        _emb_body,
        mesh=mesh,
        compiler_params=pltpu.CompilerParams(use_tc_tiling_on_sc=False),
        out_type=jax.ShapeDtypeStruct((b_total, d), jnp.float32),
        scratch_types=[
            pltpu.VMEM((_CHUNK,), jnp.int32),
            pltpu.VMEM((_CHUNK, d), jnp.float32),
            pltpu.SemaphoreType.DMA,
        ],
    )
    out = run(table, idx)
    return out.reshape(batch, hist, d)


# speedup vs baseline: 2.5332x; 1.0174x over previous
import functools

import jax
import jax.numpy as jnp
from jax import lax
from jax.experimental import pallas as pl
from jax.experimental.pallas import tpu as pltpu
from jax.experimental.pallas import tpu_sc as plsc

_NC = 2
_NS = 16
_NW = _NC * _NS

_CHUNK = 2048
_NBUF = 2


def _emb_body(table_hbm, idx_hbm, out_hbm, idx_v, rows_v,
              si0, si1, sg0, sg1, so0, so1):
    sem_i = (si0, si1)
    sem_g = (sg0, sg1)
    sem_o = (so0, so1)
    wid = lax.axis_index("s") * _NC + lax.axis_index("c")
    b_per_w = idx_hbm.shape[0] // _NW
    base = wid * b_per_w
    nchunks = b_per_w // _CHUNK

    def idx_copy(c, b):
        return pltpu.make_async_copy(
            idx_hbm.at[pl.ds(base + c * _CHUNK, _CHUNK)], idx_v.at[b], sem_i[b])

    def gather(b):
        return pltpu.make_async_copy(
            table_hbm.at[idx_v.at[b]], rows_v.at[b], sem_g[b])

    def out_copy(c, b):
        return pltpu.make_async_copy(
            rows_v.at[b], out_hbm.at[pl.ds(base + c * _CHUNK, _CHUNK)], sem_o[b])

    for b in range(_NBUF):
        idx_copy(b, b).start()

    def step(g2, carry):
        g = g2 * _NBUF
        for b in range(_NBUF):
            c = g + b
            idx_copy(c, b).wait()

            @pl.when(g >= _NBUF)
            def _():
                out_copy(c - _NBUF, b).wait()

            gather(b).start()
            gather(b).wait()
            out_copy(c, b).start()

            @pl.when(c + _NBUF < nchunks)
            def _():
                idx_copy(c + _NBUF, b).start()
        return carry

    lax.fori_loop(0, nchunks // _NBUF, step, 0)

    for b in range(_NBUF):
        out_copy(nchunks - _NBUF + b, b).wait()


def kernel(e, table):
    batch, hist = e.shape
    vocab, d = table.shape
    b_total = batch * hist
    assert b_total % (_NW * _CHUNK * _NBUF) == 0

    idx = e.reshape(b_total).astype(jnp.int32)

    mesh = plsc.VectorSubcoreMesh(core_axis_name="c", subcore_axis_name="s")
    run = pl.kernel(
        _emb_body,
        mesh=mesh,
        compiler_params=pltpu.CompilerParams(use_tc_tiling_on_sc=False),
        out_type=jax.ShapeDtypeStruct((b_total, d), jnp.float32),
        scratch_types=[
            pltpu.VMEM((_NBUF, _CHUNK), jnp.int32),
            pltpu.VMEM((_NBUF, _CHUNK, d), jnp.float32),
        ] + [pltpu.SemaphoreType.DMA] * 6,
    )
    out = run(table, idx)
    return out.reshape(batch, hist, d)


# speedup vs baseline: 3.5447x; 1.3993x over previous
import functools

import jax
import jax.numpy as jnp
from jax import lax
from jax.experimental import pallas as pl
from jax.experimental.pallas import tpu as pltpu
from jax.experimental.pallas import tpu_sc as plsc

_NC = 2
_NS = 16
_NW = _NC * _NS

_BW = 512
_TILES = _BW // 128
_NBUF = 2


def _emb_body(table_hbm, idx_hbm, out_hbm, idx_v, rows_v, y_v,
              si0, si1, sg0, sg1, so0, so1):
    sem_i = (si0, si1)
    sem_g = (sg0, sg1)
    sem_o = (so0, so1)
    wid = lax.axis_index("s") * _NC + lax.axis_index("c")
    b0 = wid * _BW
    j0 = wid * _TILES
    nsteps = out_hbm.shape[0]

    def idx_copy(h, b):
        return pltpu.make_async_copy(
            idx_hbm.at[pl.ds(h * 16384 + b0, _BW)], idx_v.at[b], sem_i[b])

    def gather(b):
        return pltpu.make_async_copy(
            table_hbm.at[idx_v.at[b]], rows_v.at[b], sem_g[b])

    def out_copy(h, b, d):
        return pltpu.make_async_copy(
            y_v.at[b, d],
            out_hbm.at[h, d // 8, pl.ds(j0, _TILES), d % 8, :],
            sem_o[b])

    def transpose(b):
        rowsb = rows_v.at[b]
        iota = lax.iota(jnp.int32, 16)
        for d in range(16):
            col = jnp.full((16,), d, jnp.int32)
            for tb in range(_BW // 16):
                vec = plsc.load_gather(rowsb, [iota + (tb * 16), col])
                y_v[b, d, tb // 8, pl.ds((tb % 8) * 16, 16)] = vec

    for b in range(_NBUF):
        idx_copy(b, b).start()

    def step(g2, carry):
        g = g2 * _NBUF
        for b in range(_NBUF):
            h = g + b
            idx_copy(h, b).wait()
            gather(b).start()
            gather(b).wait()

            @pl.when(g >= _NBUF)
            def _():
                for d in range(16):
                    out_copy(h - _NBUF, b, d).wait()

            transpose(b)
            for d in range(16):
                out_copy(h, b, d).start()

            @pl.when(h + _NBUF < nsteps)
            def _():
                idx_copy(h + _NBUF, b).start()
        return carry

    lax.fori_loop(0, nsteps // _NBUF, step, 0)

    for b in range(_NBUF):
        for d in range(16):
            out_copy(nsteps - _NBUF + b, b, d).wait()


def kernel(e, table):
    batch, hist = e.shape
    vocab, d = table.shape
    assert batch == _NW * _BW and d == 16

    idx = e.T.reshape(batch * hist).astype(jnp.int32)

    mesh = plsc.VectorSubcoreMesh(core_axis_name="c", subcore_axis_name="s")
    run = pl.kernel(
        _emb_body,
        mesh=mesh,
        compiler_params=pltpu.CompilerParams(use_tc_tiling_on_sc=False,
                                             needs_layout_passes=False),
        out_type=jax.ShapeDtypeStruct((hist, 2, batch // 128, 8, 128),
                                      jnp.float32),
        scratch_types=[
            pltpu.VMEM((_NBUF, _BW), jnp.int32),
            pltpu.VMEM((_NBUF, _BW, 16), jnp.float32),
            pltpu.VMEM((_NBUF, 16, _TILES, 128), jnp.float32),
        ] + [pltpu.SemaphoreType.DMA] * 6,
    )
    y6 = run(table, idx)
    return y6.transpose(2, 4, 0, 1, 3).reshape(batch, hist, d)


# speedup vs baseline: 3.9808x; 1.1230x over previous
import functools

import jax
import jax.numpy as jnp
from jax import lax
from jax.experimental import pallas as pl
from jax.experimental.pallas import tpu as pltpu
from jax.experimental.pallas import tpu_sc as plsc

_NC = 2
_NS = 16
_NW = _NC * _NS

_BW = 512
_TILES = _BW // 128
_NBUF = 2
_PITCH = 17


def _emb_body(table_hbm, idx_hbm, out_hbm, idx_v, rows_v, rp_v, y_v,
              si0, si1, sg0, sg1, so0, so1):
    sem_i = (si0, si1)
    sem_g = (sg0, sg1)
    sem_o = (so0, so1)
    wid = lax.axis_index("s") * _NC + lax.axis_index("c")
    b0 = wid * _BW
    j0 = wid * _TILES
    nsteps = out_hbm.shape[0]

    def idx_copy(h, b):
        return pltpu.make_async_copy(
            idx_hbm.at[pl.ds(h * 16384 + b0, _BW)], idx_v.at[b], sem_i[b])

    def gather(b):
        return pltpu.make_async_copy(
            table_hbm.at[idx_v.at[b]], rows_v.at[b], sem_g[b])

    def out_copy(h, b, d):
        return pltpu.make_async_copy(
            y_v.at[b, d],
            out_hbm.at[h, d // 8, pl.ds(j0, _TILES), d % 8, :],
            sem_o[b])

    def transpose(b):
        iota17 = lax.iota(jnp.int32, 16) * _PITCH
        for t in range(_BW):
            rp_v[b, pl.ds(t * _PITCH, 16)] = rows_v[b, t, :]
        for d in range(16):
            for tb in range(_BW // 16):
                vec = plsc.load_gather(
                    rp_v.at[b], [iota17 + (tb * 16 * _PITCH + d)])
                y_v[b, d, tb // 8, pl.ds((tb % 8) * 16, 16)] = vec

    for b in range(_NBUF):
        idx_copy(b, b).start()

    def step(g2, carry):
        g = g2 * _NBUF
        for b in range(_NBUF):
            h = g + b
            idx_copy(h, b).wait()
            gather(b).start()
            gather(b).wait()

            @pl.when(g >= _NBUF)
            def _():
                for d in range(16):
                    out_copy(h - _NBUF, b, d).wait()

            transpose(b)
            for d in range(16):
                out_copy(h, b, d).start()

            @pl.when(h + _NBUF < nsteps)
            def _():
                idx_copy(h + _NBUF, b).start()
        return carry

    lax.fori_loop(0, nsteps // _NBUF, step, 0)

    for b in range(_NBUF):
        for d in range(16):
            out_copy(nsteps - _NBUF + b, b, d).wait()


def kernel(e, table):
    batch, hist = e.shape
    vocab, d = table.shape
    assert batch == _NW * _BW and d == 16

    idx = e.T.reshape(batch * hist).astype(jnp.int32)

    mesh = plsc.VectorSubcoreMesh(core_axis_name="c", subcore_axis_name="s")
    run = pl.kernel(
        _emb_body,
        mesh=mesh,
        compiler_params=pltpu.CompilerParams(use_tc_tiling_on_sc=False,
                                             needs_layout_passes=False),
        out_type=jax.ShapeDtypeStruct((hist, 2, batch // 128, 8, 128),
                                      jnp.float32),
        scratch_types=[
            pltpu.VMEM((_NBUF, _BW), jnp.int32),
            pltpu.VMEM((_NBUF, _BW, 16), jnp.float32),
            pltpu.VMEM((_NBUF, _BW * _PITCH), jnp.float32),
            pltpu.VMEM((_NBUF, 16, _TILES, 128), jnp.float32),
        ] + [pltpu.SemaphoreType.DMA] * 6,
    )
    y6 = run(table, idx)
    return y6.transpose(2, 4, 0, 1, 3).reshape(batch, hist, d)


# speedup vs baseline: 4.5882x; 1.1526x over previous
import functools

import jax
import jax.numpy as jnp
from jax import lax
from jax.experimental import pallas as pl
from jax.experimental.pallas import tpu as pltpu
from jax.experimental.pallas import tpu_sc as plsc

_NC = 2
_NS = 16
_NW = _NC * _NS

_BW = 512
_TILES = _BW // 128
_NBUF = 2
_PITCH = 17


def _emb_body(table_hbm, idx_hbm, out_hbm, idx_v, rows_v, rp_v, y_v,
              si0, si1, sg0, sg1, so0, so1):
    sem_i = (si0, si1)
    sem_g = (sg0, sg1)
    sem_o = (so0, so1)
    wid = lax.axis_index("s") * _NC + lax.axis_index("c")
    b0 = wid * _BW
    j0 = wid * _TILES
    nsteps = out_hbm.shape[0]

    def idx_copy(h, b):
        return pltpu.make_async_copy(
            idx_hbm.at[pl.ds(h * 16384 + b0, _BW)], idx_v.at[b], sem_i[b])

    def gather(b):
        return pltpu.make_async_copy(
            table_hbm.at[idx_v.at[b]], rows_v.at[b], sem_g[b])

    def out_copy(h, b, d):
        return pltpu.make_async_copy(
            y_v.at[b, d],
            out_hbm.at[h, d // 8, pl.ds(j0, _TILES), d % 8, :],
            sem_o[b])

    def transpose(b):
        iota17 = lax.iota(jnp.int32, 16) * _PITCH
        for t in range(_BW):
            rp_v[b, pl.ds(t * _PITCH, 16)] = rows_v[b, t, :]
        for d in range(16):
            for tb in range(_BW // 16):
                vec = plsc.load_gather(
                    rp_v.at[b], [iota17 + (tb * 16 * _PITCH + d)])
                y_v[b, d, tb // 8, pl.ds((tb % 8) * 16, 16)] = vec

    for b in range(_NBUF):
        idx_copy(b, b).start()
    idx_copy(0, 0).wait()
    gather(0).start()

    def step(g2, carry):
        for k in range(_NBUF):
            h = g2 * _NBUF + k
            b = k
            b1 = 1 - k
            gather(b).wait()

            @pl.when(h + 1 < nsteps)
            def _():
                idx_copy(h + 1, b1).wait()
                gather(b1).start()

            @pl.when(h + 2 < nsteps)
            def _():
                idx_copy(h + 2, b).start()

            @pl.when(h >= _NBUF)
            def _():
                for d in range(16):
                    out_copy(h - _NBUF, b, d).wait()

            transpose(b)
            for d in range(16):
                out_copy(h, b, d).start()
        return carry

    lax.fori_loop(0, nsteps // _NBUF, step, 0)

    for b in range(_NBUF):
        for d in range(16):
            out_copy(nsteps - _NBUF + b, b, d).wait()


def kernel(e, table):
    batch, hist = e.shape
    vocab, d = table.shape
    assert batch == _NW * _BW and d == 16

    idx = e.T.reshape(batch * hist).astype(jnp.int32)

    mesh = plsc.VectorSubcoreMesh(core_axis_name="c", subcore_axis_name="s")
    run = pl.kernel(
        _emb_body,
        mesh=mesh,
        compiler_params=pltpu.CompilerParams(use_tc_tiling_on_sc=False,
                                             needs_layout_passes=False),
        out_type=jax.ShapeDtypeStruct((hist, 2, batch // 128, 8, 128),
                                      jnp.float32),
        scratch_types=[
            pltpu.VMEM((_NBUF, _BW), jnp.int32),
            pltpu.VMEM((_NBUF, _BW, 16), jnp.float32),
            pltpu.VMEM((_NBUF, _BW * _PITCH), jnp.float32),
            pltpu.VMEM((_NBUF, 16, _TILES, 128), jnp.float32),
        ] + [pltpu.SemaphoreType.DMA] * 6,
    )
    y6 = run(table, idx)
    return y6.transpose(2, 4, 0, 1, 3).reshape(batch, hist, d)


# speedup vs baseline: 8.1572x; 1.7779x over previous
import functools

import jax
import jax.numpy as jnp
from jax import lax
from jax.experimental import pallas as pl
from jax.experimental.pallas import tpu as pltpu
from jax.experimental.pallas import tpu_sc as plsc

_NC = 2
_NS = 16
_NW = _NC * _NS

_BW = 512
_TILES = _BW // 128
_NBUF = 2
_PITCH = 17


def _emb_body(table_hbm, idx_hbm, out_hbm, idx_v, rows_v, rp_v, y_v,
              si0, si1, sg0, sg1, so0, so1):
    sem_i = (si0, si1)
    sem_g = (sg0, sg1)
    sem_o = (so0, so1)
    wid = lax.axis_index("s") * _NC + lax.axis_index("c")
    b0 = wid * _BW
    j0 = wid * _TILES
    nsteps = out_hbm.shape[0]

    def idx_copy(h, b):
        return pltpu.make_async_copy(
            idx_hbm.at[pl.ds(h * 16384 + b0, _BW)], idx_v.at[b], sem_i[b])

    def gather(b):
        return pltpu.make_async_copy(
            table_hbm.at[idx_v.at[b]], rows_v.at[b], sem_g[b])

    def out_copy(h, b, d):
        return pltpu.make_async_copy(
            y_v.at[b, d],
            out_hbm.at[h, d // 8, pl.ds(j0, _TILES), d % 8, :],
            sem_o[b])

    def transpose(b):
        iota17 = lax.iota(jnp.int32, 16) * _PITCH

        @plsc.parallel_loop(0, _BW, 1, unroll=16)
        def _repack(t):
            rp_v[b, pl.ds(t * _PITCH, 16)] = rows_v[b, t, :]

        @plsc.parallel_loop(0, _BW, 1, unroll=16)
        def _col(i):
            d = i & 15
            tb = i >> 4
            vec = plsc.load_gather(
                rp_v.at[b], [iota17 + (tb * (16 * _PITCH) + d)])
            y_v[b, d, tb >> 3, pl.ds((tb & 7) * 16, 16)] = vec

    for b in range(_NBUF):
        idx_copy(b, b).start()
    idx_copy(0, 0).wait()
    gather(0).start()

    def step(g2, carry):
        for k in range(_NBUF):
            h = g2 * _NBUF + k
            b = k
            b1 = 1 - k
            gather(b).wait()

            @pl.when(h + 1 < nsteps)
            def _():
                idx_copy(h + 1, b1).wait()
                gather(b1).start()

            @pl.when(h + 2 < nsteps)
            def _():
                idx_copy(h + 2, b).start()

            @pl.when(h >= _NBUF)
            def _():
                for d in range(16):
                    out_copy(h - _NBUF, b, d).wait()

            transpose(b)
            for d in range(16):
                out_copy(h, b, d).start()
        return carry

    lax.fori_loop(0, nsteps // _NBUF, step, 0)

    for b in range(_NBUF):
        for d in range(16):
            out_copy(nsteps - _NBUF + b, b, d).wait()


def kernel(e, table):
    batch, hist = e.shape
    vocab, d = table.shape
    assert batch == _NW * _BW and d == 16

    idx = e.T.reshape(batch * hist).astype(jnp.int32)

    mesh = plsc.VectorSubcoreMesh(core_axis_name="c", subcore_axis_name="s")
    run = pl.kernel(
        _emb_body,
        mesh=mesh,
        compiler_params=pltpu.CompilerParams(use_tc_tiling_on_sc=False,
                                             needs_layout_passes=False),
        out_type=jax.ShapeDtypeStruct((hist, 2, batch // 128, 8, 128),
                                      jnp.float32),
        scratch_types=[
            pltpu.VMEM((_NBUF, _BW), jnp.int32),
            pltpu.VMEM((_NBUF, _BW, 16), jnp.float32),
            pltpu.VMEM((_NBUF, _BW * _PITCH), jnp.float32),
            pltpu.VMEM((_NBUF, 16, _TILES, 128), jnp.float32),
        ] + [pltpu.SemaphoreType.DMA] * 6,
    )
    y6 = run(table, idx)
    return y6.transpose(2, 4, 0, 1, 3).reshape(batch, hist, d)
